# Initial kernel scaffold; baseline (speedup 1.0000x reference)
import jax
import jax.numpy as jnp
from jax.experimental import pallas as pl


def kernel(x, hyperedge_index, W1, b1, W2, b2, Wds, bds, Wlin, blin):
    raise NotImplementedError("write your pallas kernel here")


# speedup vs baseline: 1.2588x; 1.2588x over previous
import functools

import jax
import jax.numpy as jnp
from jax import lax
from jax.experimental import pallas as pl
from jax.experimental.pallas import tpu as pltpu
from jax.experimental.pallas import tpu_sc as plsc

NN = 10000
NE = 5000
DIM = 256
NNZ = 160000
NC = 2
NS = 16
G = 640
RPT = G // NS
ROUNDS = NNZ // NC // G
HCH = 2000
HPT = NNZ // NS
ROWB = 1000

_MESH = plsc.VectorSubcoreMesh(core_axis_name="c", subcore_axis_name="s",
                               num_cores=NC)
_NT = pltpu.CompilerParams(use_tc_tiling_on_sc=False)


@functools.partial(
    pl.kernel, mesh=_MESH, compiler_params=_NT,
    out_type=jax.ShapeDtypeStruct((NC, NS * NE, 16), jnp.float32),
    scratch_types=[
        pltpu.VMEM((HCH,), jnp.int32),
        pltpu.VMEM((NE, 16), jnp.float32),
        pltpu.VMEM((16, 16), jnp.float32),
    ],
)
def _hist(idx2_hbm, ones_hbm, out_hbm, idx_v, hist_v, ones_v):
    c = lax.axis_index("c")
    s = lax.axis_index("s")
    zero16 = jnp.zeros((16,), jnp.float32)
    pltpu.sync_copy(ones_hbm, ones_v)

    def zrow(i, carry):
        hist_v[i, :] = zero16
        return carry

    lax.fori_loop(0, NE, zrow, 0)

    def count_chunk(ci, carry):
        base = c * NNZ + s * HPT + ci * HCH
        pltpu.sync_copy(idx2_hbm.at[pl.ds(base, HCH)], idx_v)

        def grp(g, carry2):
            vv = idx_v[pl.ds(g * 16, 16)]
            for j in range(16):
                plsc.addupdate(hist_v.at[vv[j], :], ones_v[0, :])
            return carry2

        lax.fori_loop(0, HCH // 16, grp, 0)
        return carry

    lax.fori_loop(0, HPT // HCH, count_chunk, 0)
    pltpu.sync_copy(hist_v, out_hbm.at[c, pl.ds(s * NE, NE)])


def _histred_body(dvp_ref, bep_ref, dv_ref, be_ref):
    dv_ref[...] = jnp.sum(dvp_ref[...], axis=0)
    be_ref[...] = jnp.sum(bep_ref[...], axis=0)


_hist_reduce = pl.pallas_call(
    _histred_body,
    grid=(NE // ROWB,),
    in_specs=[pl.BlockSpec((NS, ROWB, 16), lambda i: (0, i, 0)),
              pl.BlockSpec((NS, ROWB, 16), lambda i: (0, i, 0))],
    out_specs=[pl.BlockSpec((ROWB, 16), lambda i: (i, 0)),
               pl.BlockSpec((ROWB, 16), lambda i: (i, 0))],
    out_shape=[jax.ShapeDtypeStruct((NE, 16), jnp.float32),
               jax.ShapeDtypeStruct((NE, 16), jnp.float32)],
)


def _make_gs(nrows_table):
    @functools.partial(
        pl.kernel, mesh=_MESH, compiler_params=_NT,
        out_type=jax.ShapeDtypeStruct((NC, NE, DIM), jnp.float32),
        scratch_types=[
            pltpu.VMEM((RPT,), jnp.int32),
            pltpu.VMEM((RPT, DIM), jnp.float32),
            pltpu.VMEM((G,), jnp.int32),
            pltpu.VMEM((G, 16), jnp.float32),
            pltpu.VMEM((NE, 16), jnp.float32),
            pltpu.VMEM_SHARED((G, DIM), jnp.float32),
            pltpu.SemaphoreType.DMA,
        ],
    )
    def _gs(table_hbm, gidx_hbm, sidx_hbm, out_hbm,
            gi_v, rows_v, si_v, cols_v, acc_v, stage_sh, sem):
        c = lax.axis_index("c")
        s = lax.axis_index("s")
        zero16 = jnp.zeros((16,), jnp.float32)

        def zrow(i, carry):
            acc_v[i, :] = zero16
            return carry

        lax.fori_loop(0, NE, zrow, 0)

        def rnd(r, carry):
            base = c * (NNZ // NC) + r * G
            pltpu.sync_copy(gidx_hbm.at[pl.ds(base + s * RPT, RPT)], gi_v)
            pltpu.async_copy(table_hbm.at[gi_v], rows_v, sem).wait()
            pltpu.sync_copy(rows_v, stage_sh.at[pl.ds(s * RPT, RPT)])
            plsc.subcore_barrier()
            pltpu.sync_copy(stage_sh.at[:, pl.ds(s * 16, 16)], cols_v)
            plsc.subcore_barrier()
            pltpu.sync_copy(sidx_hbm.at[pl.ds(base, G)], si_v)

            def grp(g, carry2):
                sv = si_v[pl.ds(g * 16, 16)]
                for j in range(16):
                    plsc.addupdate(acc_v.at[sv[j], :], cols_v[g * 16 + j, :])
                return carry2

            lax.fori_loop(0, G // 16, grp, 0)
            return carry

        lax.fori_loop(0, ROUNDS, rnd, 0)
        pltpu.sync_copy(acc_v, out_hbm.at[c, :, pl.ds(s * 16, 16)])

    return _gs


_gs10 = _make_gs(NN)
_gs5 = _make_gs(NE)


def _mm2_body(x_ref, w1_ref, wds_ref, bds_ref, xw_ref, org_ref):
    xv = x_ref[...]
    xw_ref[...] = jnp.dot(xv, w1_ref[...], preferred_element_type=jnp.float32)
    org_ref[...] = (jnp.dot(xv, wds_ref[...],
                            preferred_element_type=jnp.float32) + bds_ref[...])


_mm2 = pl.pallas_call(
    _mm2_body,
    grid=(NN // ROWB,),
    in_specs=[pl.BlockSpec((ROWB, DIM), lambda i: (i, 0)),
              pl.BlockSpec((DIM, DIM), lambda i: (0, 0)),
              pl.BlockSpec((DIM, DIM), lambda i: (0, 0)),
              pl.BlockSpec((1, DIM), lambda i: (0, 0))],
    out_specs=[pl.BlockSpec((ROWB, DIM), lambda i: (i, 0)),
               pl.BlockSpec((ROWB, DIM), lambda i: (i, 0))],
    out_shape=[jax.ShapeDtypeStruct((NN, DIM), jnp.float32),
               jax.ShapeDtypeStruct((NN, DIM), jnp.float32)],
)


def _inv_counts(deg_ref):
    cnt = deg_ref[:, 0:1]
    return jnp.where(cnt > 0, 1.0 / cnt, 0.0)


def _comb_body(p_ref, be_ref, m_ref):
    m_ref[...] = (p_ref[0] + p_ref[1]) * _inv_counts(be_ref)


_combine = pl.pallas_call(
    _comb_body,
    grid=(NE // ROWB,),
    in_specs=[pl.BlockSpec((2, ROWB, DIM), lambda i: (0, i, 0)),
              pl.BlockSpec((ROWB, 16), lambda i: (i, 0))],
    out_specs=pl.BlockSpec((ROWB, DIM), lambda i: (i, 0)),
    out_shape=jax.ShapeDtypeStruct((NE, DIM), jnp.float32),
)


def _actmm_body(q_ref, dv_ref, b1_ref, w2_ref, xw2_ref):
    h = jnp.tanh((q_ref[0] + q_ref[1]) * _inv_counts(dv_ref) + b1_ref[...])
    xw2_ref[...] = jnp.dot(h, w2_ref[...], preferred_element_type=jnp.float32)


_act_mm = pl.pallas_call(
    _actmm_body,
    grid=(NE // ROWB,),
    in_specs=[pl.BlockSpec((2, ROWB, DIM), lambda i: (0, i, 0)),
              pl.BlockSpec((ROWB, 16), lambda i: (i, 0)),
              pl.BlockSpec((1, DIM), lambda i: (0, 0)),
              pl.BlockSpec((DIM, DIM), lambda i: (0, 0))],
    out_specs=pl.BlockSpec((ROWB, DIM), lambda i: (i, 0)),
    out_shape=jax.ShapeDtypeStruct((NE, DIM), jnp.float32),
)


def _final_body(q_ref, dv_ref, b2_ref, org_ref, wl_ref, bl_ref, o_ref):
    i = pl.program_id(0)
    row = i * ROWB + lax.broadcasted_iota(jnp.int32, (ROWB, 1), 0)
    cnt = dv_ref[:, 0:1]
    inv = jnp.where((row < NE) & (cnt > 0), 1.0 / cnt, 0.0)
    h = jnp.tanh((q_ref[0] + q_ref[1]) * inv + b2_ref[...])
    h = h + org_ref[...]
    logits = jnp.dot(h, wl_ref[...], preferred_element_type=jnp.float32)
    logits = logits + bl_ref[...]
    mx = jnp.max(logits, axis=-1, keepdims=True)
    lse = jnp.log(jnp.sum(jnp.exp(logits - mx), axis=-1, keepdims=True)) + mx
    o_ref[...] = logits - lse


_final = pl.pallas_call(
    _final_body,
    grid=(NN // ROWB,),
    in_specs=[pl.BlockSpec((2, ROWB, DIM), lambda i: (0, jnp.minimum(i, 4), 0)),
              pl.BlockSpec((ROWB, 16), lambda i: (jnp.minimum(i, 4), 0)),
              pl.BlockSpec((1, DIM), lambda i: (0, 0)),
              pl.BlockSpec((ROWB, DIM), lambda i: (i, 0)),
              pl.BlockSpec((DIM, 40), lambda i: (0, 0)),
              pl.BlockSpec((1, 40), lambda i: (0, 0))],
    out_specs=pl.BlockSpec((ROWB, 40), lambda i: (i, 0)),
    out_shape=jax.ShapeDtypeStruct((NN, 40), jnp.float32),
)


def kernel(x, hyperedge_index, W1, b1, W2, b2, Wds, bds, Wlin, blin):
    nidx = hyperedge_index[0].astype(jnp.int32)
    eidx = hyperedge_index[1].astype(jnp.int32)

    xw1, origin = _mm2(x, W1, Wds, bds.reshape(1, DIM))
    one0_tab = jnp.tile((jnp.arange(16) == 0).astype(jnp.float32), (16, 1))
    hp = _hist(jnp.concatenate([nidx, eidx]), one0_tab)
    dv_tab, be_tab = _hist_reduce(hp[0].reshape(NS, NE, 16),
                                  hp[1].reshape(NS, NE, 16))

    p1 = _gs10(xw1, nidx, eidx)
    m1 = _combine(p1, be_tab)
    q1 = _gs5(m1, eidx, nidx)
    xw2 = _act_mm(q1, dv_tab, b1.reshape(1, DIM), W2)
    p2 = _gs5(xw2, nidx, eidx)
    m2 = _combine(p2, be_tab)
    q2 = _gs5(m2, eidx, nidx)
    return _final(q2, dv_tab, b2.reshape(1, DIM), origin, Wlin,
                  blin.reshape(1, 40))
